# baseline (speedup 1.0000x reference)
import functools

import jax
import jax.numpy as jnp
from jax import lax
from jax.experimental import pallas as pl
from jax.experimental.pallas import tpu as pltpu
from jax.experimental.pallas import tpu_sc as plsc

_NUM_TASKS = 512
_MAX_COUNT_CAP = 64
_LAG_WEEKS = 4
_CONFIDENCE_LOGIT = 12.0
_OFF_LOGIT = -12.0

_K = _MAX_COUNT_CAP + 1
_BATCH = 4096
_HALF = 256
_NBLOCKS = _BATCH * _NUM_TASKS // _HALF
_NW = 32
_BLOCKS_PER_W = _NBLOCKS // _NW
_L = 16
_MAGIC = 12582912.0


def _fill_block(buf, value):
    vec = jnp.full((_L,), value, jnp.float32)
    lanes = lax.iota(jnp.int32, _L)

    def _row(r, carry):
        rv = jnp.zeros((_L,), jnp.int32) + r
        for c in range(5):
            cv = jnp.minimum(lanes + c * _L, _MAX_COUNT_CAP)
            plsc.store_scatter(buf, [rv, cv], vec)
        return carry

    lax.fori_loop(0, _HALF, _row, 0)


def _sc_body(lag_hbm, out_hbm, buf0, buf1, idx0, idx1, lagv, sem0, sem1):
    wid = lax.axis_index("s") * 2 + lax.axis_index("c")
    base = wid * _BLOCKS_PER_W

    _fill_block(buf0, _OFF_LOGIT)
    _fill_block(buf1, _OFF_LOGIT)

    lanes = lax.iota(jnp.int32, _L)
    hit = jnp.full((_L,), _CONFIDENCE_LOGIT, jnp.float32)
    off = jnp.full((_L,), _OFF_LOGIT, jnp.float32)

    def _dst(h):
        return out_hbm.at[pl.ds((base + h) * _HALF, _HALF)]

    def _process(h, buf, idx, sem):
        @pl.when(h >= 2)
        def _():
            pltpu.make_async_copy(buf, _dst(h - 2), sem).wait()
            for c in range(_HALF // _L):
                tv = lanes + c * _L
                cv = idx[pl.ds(c * _L, _L)]
                plsc.store_scatter(buf, [tv, cv], off)

        pltpu.sync_copy(lag_hbm.at[pl.ds(base + h, 1)], lagv)
        for c in range(_HALF // _L):
            x = lagv[0, pl.ds(c * _L, _L)]
            y = jnp.minimum(
                jnp.maximum((x + _MAGIC) - _MAGIC, 0.0),
                float(_MAX_COUNT_CAP),
            )
            cv = y.astype(jnp.int32)
            idx[pl.ds(c * _L, _L)] = cv
            tv = lanes + c * _L
            plsc.store_scatter(buf, [tv, cv], hit)
        pltpu.async_copy(buf, _dst(h), sem)

    def _step(i, carry):
        _process(2 * i, buf0, idx0, sem0)
        _process(2 * i + 1, buf1, idx1, sem1)
        return carry

    lax.fori_loop(0, _BLOCKS_PER_W // 2, _step, 0)
    pltpu.make_async_copy(buf0, _dst(_BLOCKS_PER_W - 2), sem0).wait()
    pltpu.make_async_copy(buf1, _dst(_BLOCKS_PER_W - 1), sem1).wait()


@jax.jit
def kernel(sequence):
    batch_size, window_size, _ = sequence.shape
    lag = sequence[:, window_size - _LAG_WEEKS, :_NUM_TASKS]
    lag2 = lag.reshape(batch_size * _NUM_TASKS // _HALF, _HALF)
    mesh = plsc.VectorSubcoreMesh(core_axis_name="c", subcore_axis_name="s")
    sc = functools.partial(
        pl.kernel,
        mesh=mesh,
        compiler_params=pltpu.CompilerParams(
            needs_layout_passes=False, use_tc_tiling_on_sc=False
        ),
        out_type=jax.ShapeDtypeStruct(
            (batch_size * _NUM_TASKS, _K), jnp.float32
        ),
        scratch_types=[
            pltpu.VMEM((_HALF, _K), jnp.float32),
            pltpu.VMEM((_HALF, _K), jnp.float32),
            pltpu.VMEM((_HALF,), jnp.int32),
            pltpu.VMEM((_HALF,), jnp.int32),
            pltpu.VMEM((1, _HALF), jnp.float32),
            pltpu.SemaphoreType.DMA,
            pltpu.SemaphoreType.DMA,
        ],
    )(_sc_body)
    flat = sc(lag2)
    return flat.reshape(batch_size, _NUM_TASKS, _K)


# speedup vs baseline: 2.0559x; 2.0559x over previous
import functools

import jax
import jax.numpy as jnp
from jax import lax
from jax.experimental import pallas as pl
from jax.experimental.pallas import tpu as pltpu
from jax.experimental.pallas import tpu_sc as plsc

_NUM_TASKS = 512
_MAX_COUNT_CAP = 64
_LAG_WEEKS = 4
_CONFIDENCE_LOGIT = 12.0
_OFF_LOGIT = -12.0

_K = _MAX_COUNT_CAP + 1
_BATCH = 4096
_BLK = 128
_NBLOCKS = _BATCH * _NUM_TASKS // _BLK
_NW = 32
_BLOCKS_PER_W = _NBLOCKS // _NW
_NBUF = 4
_L = 16
_MAGIC = 12582912.0


def _fill_block(buf, value):
    vec = jnp.full((_L,), value, jnp.float32)
    lanes = lax.iota(jnp.int32, _L)

    def _row(r, carry):
        rv = jnp.zeros((_L,), jnp.int32) + r
        for c in range(5):
            cv = jnp.minimum(lanes + c * _L, _MAX_COUNT_CAP)
            plsc.store_scatter(buf, [rv, cv], vec)
        return carry

    lax.fori_loop(0, _BLK, _row, 0)


def _sc_body(
    lag_hbm,
    out_hbm,
    bufs,
    idxs,
    lags,
    osems,
    lsems,
):
    wid = lax.axis_index("s") * 2 + lax.axis_index("c")
    base = wid * _BLOCKS_PER_W

    for s in range(_NBUF):
        _fill_block(bufs[s], _OFF_LOGIT)

    lanes = lax.iota(jnp.int32, _L)
    hit = jnp.full((_L,), _CONFIDENCE_LOGIT, jnp.float32)
    off = jnp.full((_L,), _OFF_LOGIT, jnp.float32)

    def _dst(h):
        return out_hbm.at[pl.ds((base + h) * _BLK, _BLK)]

    def _lag_fetch(h, s):
        return pltpu.async_copy(
            lag_hbm.at[pl.ds(base + h, 1)], lags[s], lsems[s]
        )

    for s in range(_NBUF):
        _lag_fetch(s, s)

    def _process(h, s):
        buf, idx, lag = bufs[s], idxs[s], lags[s]

        @pl.when(h >= _NBUF)
        def _():
            pltpu.make_async_copy(buf, _dst(h - _NBUF), osems[s]).wait()

        pltpu.make_async_copy(
            lag_hbm.at[pl.ds(base + h, 1)], lag, lsems[s]
        ).wait()

        for c in range(_BLK // _L):
            tv = lanes + c * _L
            sl = pl.ds(c * _L, _L)

            @pl.when(h >= _NBUF)
            def _():
                plsc.store_scatter(buf, [tv, idx[sl]], off)

            x = lag[0, sl]
            y = jnp.minimum(
                jnp.maximum((x + _MAGIC) - _MAGIC, 0.0),
                float(_MAX_COUNT_CAP),
            )
            cv = y.astype(jnp.int32)
            idx[sl] = cv
            plsc.store_scatter(buf, [tv, cv], hit)

        pltpu.async_copy(buf, _dst(h), osems[s])

        @pl.when(h + _NBUF < _BLOCKS_PER_W)
        def _():
            _lag_fetch(h + _NBUF, s)

    def _step(i, carry):
        for s in range(_NBUF):
            _process(_NBUF * i + s, s)
        return carry

    lax.fori_loop(0, _BLOCKS_PER_W // _NBUF, _step, 0)
    for s in range(_NBUF):
        pltpu.make_async_copy(
            bufs[s], _dst(_BLOCKS_PER_W - _NBUF + s), osems[s]
        ).wait()


@jax.jit
def kernel(sequence):
    batch_size, window_size, _ = sequence.shape
    lag = sequence[:, window_size - _LAG_WEEKS, :_NUM_TASKS]
    lag2 = lag.reshape(batch_size * _NUM_TASKS // _BLK, _BLK)
    mesh = plsc.VectorSubcoreMesh(core_axis_name="c", subcore_axis_name="s")

    def body(lag_hbm, out_hbm, *scratch):
        bufs = scratch[0:_NBUF]
        idxs = scratch[_NBUF : 2 * _NBUF]
        lags = scratch[2 * _NBUF : 3 * _NBUF]
        osems = scratch[3 * _NBUF : 4 * _NBUF]
        lsems = scratch[4 * _NBUF : 5 * _NBUF]
        _sc_body(lag_hbm, out_hbm, bufs, idxs, lags, osems, lsems)

    sc = functools.partial(
        pl.kernel,
        mesh=mesh,
        compiler_params=pltpu.CompilerParams(needs_layout_passes=False),
        out_type=jax.ShapeDtypeStruct(
            (batch_size * _NUM_TASKS, _K), jnp.float32
        ),
        scratch_types=[pltpu.VMEM((_BLK, _K), jnp.float32)] * _NBUF
        + [pltpu.VMEM((_BLK,), jnp.int32)] * _NBUF
        + [pltpu.VMEM((1, _BLK), jnp.float32)] * _NBUF
        + [pltpu.SemaphoreType.DMA] * (2 * _NBUF),
    )(body)
    flat = sc(lag2)
    return flat.reshape(batch_size, _NUM_TASKS, _K)
